# baseline (speedup 1.0000x reference)
import functools

import jax
import jax.numpy as jnp
from jax import lax
from jax.experimental import pallas as pl
from jax.experimental.pallas import tpu as pltpu
from jax.experimental.pallas import tpu_sc as plsc

_L = 16
_NS = 16
_NC = 2
_B = 80
_NBUF = 25


def _matmul_body(N, x_ref, w_ref, o_ref):
    o_ref[...] = jnp.zeros_like(o_ref)
    o_ref[pl.ds(0, N)] = jnp.dot(
        x_ref[...], w_ref[...], preferred_element_type=jnp.float32)


def _final_body(p2_ref, w_ref, b_ref, o_ref):
    s = p2_ref[0] + p2_ref[1]
    logits = jnp.dot(s, w_ref[...], preferred_element_type=jnp.float32)
    logits = logits + b_ref[...]
    m = jnp.max(logits, axis=1, keepdims=True)
    t = logits - m
    lse = jnp.log(jnp.sum(jnp.exp(t), axis=1, keepdims=True))
    o_ref[...] = t - lse


def _rsqrt16(x):
    i = plsc.bitcast(x, jnp.int32)
    y = plsc.bitcast(jnp.int32(0x5F3759DF) - (i >> 1), jnp.float32)
    for _ in range(3):
        y = y * (1.5 - 0.5 * x * y * y)
    return y


def _edge_pass(tbl_hbm, srcb, dstb, rows, sems, accum, nchunks):
    def _round(j, c):
        i0 = j * _NBUF
        handles = [
            pltpu.async_copy(tbl_hbm.at[srcb.at[i0 + b]], rows.at[b], sems[b])
            for b in range(_NBUF)
        ]
        for b in range(_NBUF):
            handles[b].wait()
            pltpu.sync_copy(rows.at[b], accum.at[dstb.at[i0 + b]], add=True)
        return c

    lax.fori_loop(0, nchunks // _NBUF, _round, 0)


@functools.lru_cache(maxsize=None)
def _make_k1(N, E, Np):
    EC = E // (_NC * _NS)
    RH = EC // _B
    NPT = Np // _NS
    mesh = plsc.VectorSubcoreMesh(
        core_axis_name="c", subcore_axis_name="s", num_cores=_NC)

    @functools.partial(
        pl.kernel,
        out_type=[jax.ShapeDtypeStruct((_NC, Np), jnp.int32)],
        mesh=mesh,
        compiler_params=pltpu.CompilerParams(
            needs_layout_passes=False, use_tc_tiling_on_sc=False),
        scratch_types=[
            pltpu.VMEM((RH, _B), jnp.int32),
            pltpu.VMEM((Np,), jnp.int32),
            pltpu.VMEM((_NS, NPT), jnp.int32),
            pltpu.VMEM((NPT,), jnp.int32),
            pltpu.VMEM_SHARED((_NS, Np), jnp.int32),
        ],
    )
    def k1(dst2_hbm, histo_hbm, dsth, hist, slab, racc, hist16):
        cid = lax.axis_index("c")
        sid = lax.axis_index("s")
        nbase = sid * NPT
        wid = cid * _NS + sid

        pltpu.sync_copy(dst2_hbm.at[pl.ds(wid * RH, RH)], dsth)

        def _zero(i, c):
            hist[pl.ds(i * _L, _L)] = jnp.zeros((_L,), jnp.int32)
            return c
        lax.fori_loop(0, Np // _L, _zero, 0)

        def _count(i, c):
            for t in range(_B // _L):
                d16 = dsth[i, pl.ds(t * _L, _L)]
                cnt, last = plsc.scan_count(d16)
                plsc.addupdate_scatter(hist, [d16], cnt, mask=last)
            return c
        lax.fori_loop(0, RH, _count, 0)

        pltpu.sync_copy(hist, hist16.at[sid])
        plsc.subcore_barrier()

        pltpu.sync_copy(hist16.at[:, pl.ds(nbase, NPT)], slab)

        def _racc(j, c):
            o = j * _L
            acc = slab[0, pl.ds(o, _L)]
            for r in range(1, _NS):
                acc = acc + slab[r, pl.ds(o, _L)]
            racc[pl.ds(o, _L)] = acc
            return c
        lax.fori_loop(0, NPT // _L, _racc, 0)
        pltpu.sync_copy(racc, histo_hbm.at[cid, pl.ds(nbase, NPT)])

    return k1


@functools.lru_cache(maxsize=None)
def _make_k2(N, E, H, Np):
    EC = E // (_NC * _NS)
    KCH = EC // _B
    NPT = Np // _NS
    mesh = plsc.VectorSubcoreMesh(
        core_axis_name="c", subcore_axis_name="s", num_cores=_NC)

    @functools.partial(
        pl.kernel,
        out_type=[
            jax.ShapeDtypeStruct((_NC, Np, H), jnp.float32),
            jax.ShapeDtypeStruct((_NC, Np, H), jnp.float32),
            jax.ShapeDtypeStruct((_NC, Np), jnp.float32),
        ],
        mesh=mesh,
        compiler_params=pltpu.CompilerParams(
            needs_layout_passes=False, use_tc_tiling_on_sc=False),
        scratch_types=[
            pltpu.VMEM((NPT,), jnp.int32),
            pltpu.VMEM((NPT,), jnp.float32),
            pltpu.VMEM((NPT, H), jnp.float32),
            pltpu.VMEM((NPT, H), jnp.float32),
            pltpu.VMEM((KCH, _B), jnp.int32),
            pltpu.VMEM((KCH, _B), jnp.int32),
            pltpu.VMEM((_NBUF, _B, H), jnp.float32),
            pltpu.VMEM_SHARED((Np, H), jnp.float32),
        ] + [pltpu.SemaphoreType.DMA] * _NBUF,
    )
    def k2(xw_hbm, src2_hbm, dst2_hbm, histo_hbm, ysh_hbm, part1_hbm, diso_hbm,
           rbuf, dis, bufa, bufb, srcb, dstb, rows,
           accum, *sems):
        cid = lax.axis_index("c")
        sid = lax.axis_index("s")
        nbase = sid * NPT
        sems = list(sems)

        def _deg_init(j, c):
            dis[pl.ds(j * _L, _L)] = jnp.ones((_L,), jnp.float32)
            return c
        lax.fori_loop(0, NPT // _L, _deg_init, 0)
        for r in range(_NC):
            pltpu.sync_copy(histo_hbm.at[r, pl.ds(nbase, NPT)], rbuf)

            def _deg_acc(j, c):
                o = j * _L
                dis[pl.ds(o, _L)] = (
                    dis[pl.ds(o, _L)] + rbuf[pl.ds(o, _L)].astype(jnp.float32))
                return c
            lax.fori_loop(0, NPT // _L, _deg_acc, 0)

        def _dis_blk(j, c):
            o = j * _L
            dis[pl.ds(o, _L)] = _rsqrt16(dis[pl.ds(o, _L)])
            return c
        lax.fori_loop(0, NPT // _L, _dis_blk, 0)
        pltpu.sync_copy(dis, diso_hbm.at[cid, pl.ds(nbase, NPT)])

        pltpu.sync_copy(xw_hbm.at[pl.ds(nbase, NPT)], bufa)

        def _scale_y1(j, c):
            dv = dis[pl.ds(j * _L, _L)]
            for t in range(_L):
                r = j * _L + t
                bufb[r] = bufa[r] * dv[t]
            return c
        lax.fori_loop(0, NPT // _L, _scale_y1, 0)
        pltpu.sync_copy(bufb, ysh_hbm.at[cid, pl.ds(nbase, NPT)])

        def _zrow(r, c):
            bufa[r] = jnp.zeros((_L,), jnp.float32)
            return c
        lax.fori_loop(0, NPT, _zrow, 0)
        pltpu.sync_copy(bufa, accum.at[pl.ds(nbase, NPT)])

        wid = cid * _NS + sid
        pltpu.sync_copy(src2_hbm.at[pl.ds(wid * KCH, KCH)], srcb)
        pltpu.sync_copy(dst2_hbm.at[pl.ds(wid * KCH, KCH)], dstb)
        plsc.subcore_barrier()

        _edge_pass(ysh_hbm.at[cid], srcb, dstb, rows, sems, accum, KCH)
        plsc.subcore_barrier()

        pltpu.sync_copy(accum.at[pl.ds(nbase, NPT)], bufa)
        w = (1 - cid).astype(jnp.float32)

        def _dump(r, c):
            bufa[r] = bufa[r] + bufb[r] * w
            return c
        lax.fori_loop(0, NPT, _dump, 0)
        pltpu.sync_copy(bufa, part1_hbm.at[cid, pl.ds(nbase, NPT)])

    return k2


@functools.lru_cache(maxsize=None)
def _make_k3(N, E, H, Np):
    EC = E // (_NC * _NS)
    KCH = EC // _B
    NPT = Np // _NS
    mesh = plsc.VectorSubcoreMesh(
        core_axis_name="c", subcore_axis_name="s", num_cores=_NC)

    @functools.partial(
        pl.kernel,
        out_type=[
            jax.ShapeDtypeStruct((_NC, Np, H), jnp.float32),
            jax.ShapeDtypeStruct((_NC, Np, H), jnp.float32),
        ],
        mesh=mesh,
        compiler_params=pltpu.CompilerParams(
            needs_layout_passes=False, use_tc_tiling_on_sc=False),
        scratch_types=[
            pltpu.VMEM((NPT,), jnp.float32),
            pltpu.VMEM((NPT, H), jnp.float32),
            pltpu.VMEM((NPT, H), jnp.float32),
            pltpu.VMEM((_L,), jnp.float32),
            pltpu.VMEM((KCH, _B), jnp.int32),
            pltpu.VMEM((KCH, _B), jnp.int32),
            pltpu.VMEM((_NBUF, _B, H), jnp.float32),
            pltpu.VMEM_SHARED((Np, H), jnp.float32),
        ] + [pltpu.SemaphoreType.DMA] * _NBUF,
    )
    def k3(src2_hbm, dst2_hbm, part1_hbm, ysh_hbm, diso_hbm, b1_hbm,
           ysh2_hbm, part2_hbm,
           dis, bufa, bufb, b1v, srcb, dstb, rows,
           accum, *sems):
        cid = lax.axis_index("c")
        sid = lax.axis_index("s")
        nbase = sid * NPT
        sems = list(sems)

        pltpu.sync_copy(diso_hbm.at[cid, pl.ds(nbase, NPT)], dis)
        pltpu.sync_copy(part1_hbm.at[0, pl.ds(nbase, NPT)], bufa)
        pltpu.sync_copy(part1_hbm.at[1, pl.ds(nbase, NPT)], bufb)
        pltpu.sync_copy(b1_hbm, b1v)
        b1vec = b1v[...]

        def _fin1(j, c):
            dv = dis[pl.ds(j * _L, _L)]
            for t in range(_L):
                r = j * _L + t
                s = dv[t]
                h = jnp.maximum((bufa[r] + bufb[r]) * s + b1vec, 0.0)
                bufb[r] = h * s
            return c
        lax.fori_loop(0, NPT // _L, _fin1, 0)
        pltpu.sync_copy(bufb, ysh2_hbm.at[cid, pl.ds(nbase, NPT)])

        def _zrow(r, c):
            bufa[r] = jnp.zeros((_L,), jnp.float32)
            return c
        lax.fori_loop(0, NPT, _zrow, 0)
        pltpu.sync_copy(bufa, accum.at[pl.ds(nbase, NPT)])

        wid = cid * _NS + sid
        pltpu.sync_copy(src2_hbm.at[pl.ds(wid * KCH, KCH)], srcb)
        pltpu.sync_copy(dst2_hbm.at[pl.ds(wid * KCH, KCH)], dstb)
        plsc.subcore_barrier()

        _edge_pass(ysh2_hbm.at[cid], srcb, dstb, rows, sems, accum, KCH)
        plsc.subcore_barrier()

        pltpu.sync_copy(accum.at[pl.ds(nbase, NPT)], bufa)
        w = (1 - cid).astype(jnp.float32)

        def _dump(j, c):
            dv = dis[pl.ds(j * _L, _L)]
            for t in range(_L):
                r = j * _L + t
                bufa[r] = (bufa[r] + bufb[r] * w) * dv[t]
            return c
        lax.fori_loop(0, NPT // _L, _dump, 0)
        pltpu.sync_copy(bufa, part2_hbm.at[cid, pl.ds(nbase, NPT)])

    return k3


def kernel(x, edge_index, W1, b1, W2, b2):
    N, D = x.shape
    H = W1.shape[1]
    C = W2.shape[1]
    E = edge_index.shape[1]
    Np = -(-N // (_NS * _L)) * (_NS * _L)

    src2 = edge_index[0].reshape(E // _B, _B)
    dst2 = edge_index[1].reshape(E // _B, _B)

    xw = pl.pallas_call(
        functools.partial(_matmul_body, N),
        out_shape=jax.ShapeDtypeStruct((Np, H), jnp.float32),
    )(x, W1)

    (histo,) = _make_k1(N, E, Np)(dst2)
    ysh, part1, diso = _make_k2(N, E, H, Np)(xw, src2, dst2, histo)
    ysh2, part2 = _make_k3(N, E, H, Np)(src2, dst2, part1, ysh, diso, b1)

    out = pl.pallas_call(
        _final_body,
        out_shape=jax.ShapeDtypeStruct((Np, C), jnp.float32),
    )(part2, W2, b2.reshape(1, C))
    return out[:N]


# speedup vs baseline: 1.0103x; 1.0103x over previous
import functools

import jax
import jax.numpy as jnp
from jax import lax
from jax.experimental import pallas as pl
from jax.experimental.pallas import tpu as pltpu
from jax.experimental.pallas import tpu_sc as plsc

_L = 16
_NS = 16
_NC = 2
_B = 80
_NBUF = 25


def _matmul_body(N, x_ref, w_ref, o_ref):
    o_ref[...] = jnp.zeros_like(o_ref)
    o_ref[pl.ds(0, N)] = jnp.dot(
        x_ref[...], w_ref[...], preferred_element_type=jnp.float32)


def _final_body(p2_ref, w_ref, b_ref, o_ref):
    s = p2_ref[0] + p2_ref[1]
    logits = jnp.dot(s, w_ref[...], preferred_element_type=jnp.float32)
    logits = logits + b_ref[...]
    m = jnp.max(logits, axis=1, keepdims=True)
    t = logits - m
    lse = jnp.log(jnp.sum(jnp.exp(t), axis=1, keepdims=True))
    o_ref[...] = t - lse


def _rsqrt16(x):
    i = plsc.bitcast(x, jnp.int32)
    y = plsc.bitcast(jnp.int32(0x5F3759DF) - (i >> 1), jnp.float32)
    for _ in range(3):
        y = y * (1.5 - 0.5 * x * y * y)
    return y


def _edge_pass(tbl_hbm, srcb, dstb, rows, sems, accum, nchunks):
    def _round(j, c):
        i0 = j * _NBUF
        handles = [
            pltpu.async_copy(tbl_hbm.at[srcb.at[i0 + b]], rows.at[b], sems[b])
            for b in range(_NBUF)
        ]
        for b in range(_NBUF):
            handles[b].wait()
            pltpu.sync_copy(rows.at[b], accum.at[dstb.at[i0 + b]], add=True)
        return c

    lax.fori_loop(0, nchunks // _NBUF, _round, 0)


@functools.lru_cache(maxsize=None)
def _make_k1(N, E, Np):
    EC = E // (_NC * _NS)
    RH = EC // _B
    NPT = Np // _NS
    mesh = plsc.VectorSubcoreMesh(
        core_axis_name="c", subcore_axis_name="s", num_cores=_NC)

    @functools.partial(
        pl.kernel,
        out_type=[jax.ShapeDtypeStruct((_NC, Np), jnp.int32)],
        mesh=mesh,
        compiler_params=pltpu.CompilerParams(
            needs_layout_passes=False, use_tc_tiling_on_sc=False),
        scratch_types=[
            pltpu.VMEM((RH, _B), jnp.int32),
            pltpu.VMEM((Np,), jnp.int32),
            pltpu.VMEM((_NS, NPT), jnp.int32),
            pltpu.VMEM((NPT,), jnp.int32),
            pltpu.VMEM_SHARED((_NS, Np), jnp.int32),
        ],
    )
    def k1(dst2_hbm, histo_hbm, dsth, hist, slab, racc, hist16):
        cid = lax.axis_index("c")
        sid = lax.axis_index("s")
        nbase = sid * NPT
        wid = cid * _NS + sid

        pltpu.sync_copy(dst2_hbm.at[pl.ds(wid * RH, RH)], dsth)

        def _zero(i, c):
            hist[pl.ds(i * _L, _L)] = jnp.zeros((_L,), jnp.int32)
            return c
        lax.fori_loop(0, Np // _L, _zero, 0)

        def _count(i, c):
            for t in range(_B // _L):
                d16 = dsth[i, pl.ds(t * _L, _L)]
                cnt, last = plsc.scan_count(d16)
                plsc.addupdate_scatter(hist, [d16], cnt, mask=last)
            return c
        lax.fori_loop(0, RH, _count, 0)

        pltpu.sync_copy(hist, hist16.at[sid])
        plsc.subcore_barrier()

        pltpu.sync_copy(hist16.at[:, pl.ds(nbase, NPT)], slab)

        def _racc(j, c):
            o = j * _L
            acc = slab[0, pl.ds(o, _L)]
            for r in range(1, _NS):
                acc = acc + slab[r, pl.ds(o, _L)]
            racc[pl.ds(o, _L)] = acc
            return c
        lax.fori_loop(0, NPT // _L, _racc, 0)
        pltpu.sync_copy(racc, histo_hbm.at[cid, pl.ds(nbase, NPT)])

    return k1


@functools.lru_cache(maxsize=None)
def _make_k2(N, E, H, Np):
    EC = E // (_NC * _NS)
    KCH = EC // _B
    NPT = Np // _NS
    mesh = plsc.VectorSubcoreMesh(
        core_axis_name="c", subcore_axis_name="s", num_cores=_NC)

    @functools.partial(
        pl.kernel,
        out_type=[
            jax.ShapeDtypeStruct((_NC, Np, H), jnp.float32),
            jax.ShapeDtypeStruct((_NC, Np, H), jnp.float32),
            jax.ShapeDtypeStruct((_NC, Np), jnp.float32),
        ],
        mesh=mesh,
        compiler_params=pltpu.CompilerParams(
            needs_layout_passes=False, use_tc_tiling_on_sc=False),
        scratch_types=[
            pltpu.VMEM((NPT,), jnp.int32),
            pltpu.VMEM((NPT,), jnp.float32),
            pltpu.VMEM((NPT, H), jnp.float32),
            pltpu.VMEM((NPT, H), jnp.float32),
            pltpu.VMEM((KCH, _B), jnp.int32),
            pltpu.VMEM((KCH, _B), jnp.int32),
            pltpu.VMEM((_NBUF, _B, H), jnp.float32),
            pltpu.VMEM_SHARED((Np, H), jnp.float32),
        ] + [pltpu.SemaphoreType.DMA] * _NBUF,
    )
    def k2(xw_hbm, src2_hbm, dst2_hbm, histo_hbm, ysh_hbm, part1_hbm, diso_hbm,
           rbuf, dis, bufa, bufb, srcb, dstb, rows,
           accum, *sems):
        cid = lax.axis_index("c")
        sid = lax.axis_index("s")
        nbase = sid * NPT
        sems = list(sems)

        def _deg_init(j, c):
            dis[pl.ds(j * _L, _L)] = jnp.ones((_L,), jnp.float32)
            return c
        lax.fori_loop(0, NPT // _L, _deg_init, 0)
        for r in range(_NC):
            pltpu.sync_copy(histo_hbm.at[r, pl.ds(nbase, NPT)], rbuf)

            def _deg_acc(j, c):
                o = j * _L
                dis[pl.ds(o, _L)] = (
                    dis[pl.ds(o, _L)] + rbuf[pl.ds(o, _L)].astype(jnp.float32))
                return c
            lax.fori_loop(0, NPT // _L, _deg_acc, 0)

        def _dis_blk(j, c):
            o = j * _L
            dis[pl.ds(o, _L)] = _rsqrt16(dis[pl.ds(o, _L)])
            return c
        lax.fori_loop(0, NPT // _L, _dis_blk, 0)
        pltpu.sync_copy(dis, diso_hbm.at[cid, pl.ds(nbase, NPT)])

        pltpu.sync_copy(xw_hbm.at[pl.ds(nbase, NPT)], bufa)

        def _scale_y1(j, c):
            dv = dis[pl.ds(j * _L, _L)]
            for t in range(_L):
                r = j * _L + t
                bufb[r] = bufa[r] * dv[t]
            return c
        lax.fori_loop(0, NPT // _L, _scale_y1, 0)
        pltpu.sync_copy(bufb, ysh_hbm.at[cid, pl.ds(nbase, NPT)])

        def _zrow(r, c):
            bufa[r] = jnp.zeros((_L,), jnp.float32)
            return c
        lax.fori_loop(0, NPT, _zrow, 0)
        pltpu.sync_copy(bufa, accum.at[pl.ds(nbase, NPT)])

        wid = cid * _NS + sid
        pltpu.sync_copy(src2_hbm.at[pl.ds(wid * KCH, KCH)], srcb)
        pltpu.sync_copy(dst2_hbm.at[pl.ds(wid * KCH, KCH)], dstb)
        plsc.subcore_barrier()

        _edge_pass(ysh_hbm.at[cid], srcb, dstb, rows, sems, accum, KCH)
        plsc.subcore_barrier()

        pltpu.sync_copy(accum.at[pl.ds(nbase, NPT)], bufa)
        pltpu.sync_copy(bufa, part1_hbm.at[cid, pl.ds(nbase, NPT)])

    return k2


@functools.lru_cache(maxsize=None)
def _make_k3(N, E, H, Np):
    EC = E // (_NC * _NS)
    KCH = EC // _B
    NPT = Np // _NS
    mesh = plsc.VectorSubcoreMesh(
        core_axis_name="c", subcore_axis_name="s", num_cores=_NC)

    @functools.partial(
        pl.kernel,
        out_type=[
            jax.ShapeDtypeStruct((_NC, Np, H), jnp.float32),
            jax.ShapeDtypeStruct((_NC, Np, H), jnp.float32),
        ],
        mesh=mesh,
        compiler_params=pltpu.CompilerParams(
            needs_layout_passes=False, use_tc_tiling_on_sc=False),
        scratch_types=[
            pltpu.VMEM((NPT,), jnp.float32),
            pltpu.VMEM((NPT, H), jnp.float32),
            pltpu.VMEM((NPT, H), jnp.float32),
            pltpu.VMEM((NPT, H), jnp.float32),
            pltpu.VMEM((_L,), jnp.float32),
            pltpu.VMEM((KCH, _B), jnp.int32),
            pltpu.VMEM((KCH, _B), jnp.int32),
            pltpu.VMEM((_NBUF, _B, H), jnp.float32),
            pltpu.VMEM_SHARED((Np, H), jnp.float32),
        ] + [pltpu.SemaphoreType.DMA] * _NBUF,
    )
    def k3(src2_hbm, dst2_hbm, part1_hbm, ysh_hbm, diso_hbm, b1_hbm,
           ysh2_hbm, part2_hbm,
           dis, bufa, bufb, bufy, b1v, srcb, dstb, rows,
           accum, *sems):
        cid = lax.axis_index("c")
        sid = lax.axis_index("s")
        nbase = sid * NPT
        sems = list(sems)

        pltpu.sync_copy(diso_hbm.at[cid, pl.ds(nbase, NPT)], dis)
        pltpu.sync_copy(part1_hbm.at[0, pl.ds(nbase, NPT)], bufa)
        pltpu.sync_copy(part1_hbm.at[1, pl.ds(nbase, NPT)], bufb)
        pltpu.sync_copy(ysh_hbm.at[cid, pl.ds(nbase, NPT)], bufy)
        pltpu.sync_copy(b1_hbm, b1v)
        b1vec = b1v[...]

        def _fin1(j, c):
            dv = dis[pl.ds(j * _L, _L)]
            for t in range(_L):
                r = j * _L + t
                s = dv[t]
                h = jnp.maximum((bufa[r] + bufb[r] + bufy[r]) * s + b1vec, 0.0)
                bufb[r] = h * s
            return c
        lax.fori_loop(0, NPT // _L, _fin1, 0)
        pltpu.sync_copy(bufb, ysh2_hbm.at[cid, pl.ds(nbase, NPT)])

        def _zrow(r, c):
            bufa[r] = jnp.zeros((_L,), jnp.float32)
            return c
        lax.fori_loop(0, NPT, _zrow, 0)
        pltpu.sync_copy(bufa, accum.at[pl.ds(nbase, NPT)])

        wid = cid * _NS + sid
        pltpu.sync_copy(src2_hbm.at[pl.ds(wid * KCH, KCH)], srcb)
        pltpu.sync_copy(dst2_hbm.at[pl.ds(wid * KCH, KCH)], dstb)
        plsc.subcore_barrier()

        _edge_pass(ysh2_hbm.at[cid], srcb, dstb, rows, sems, accum, KCH)
        plsc.subcore_barrier()

        pltpu.sync_copy(accum.at[pl.ds(nbase, NPT)], bufa)
        w = (1 - cid).astype(jnp.float32)

        def _dump(j, c):
            dv = dis[pl.ds(j * _L, _L)]
            for t in range(_L):
                r = j * _L + t
                bufa[r] = (bufa[r] + bufb[r] * w) * dv[t]
            return c
        lax.fori_loop(0, NPT // _L, _dump, 0)
        pltpu.sync_copy(bufa, part2_hbm.at[cid, pl.ds(nbase, NPT)])

    return k3


def kernel(x, edge_index, W1, b1, W2, b2):
    N, D = x.shape
    H = W1.shape[1]
    C = W2.shape[1]
    E = edge_index.shape[1]
    Np = -(-N // (_NS * _L)) * (_NS * _L)

    src2 = edge_index[0].reshape(E // _B, _B)
    dst2 = edge_index[1].reshape(E // _B, _B)

    xw = pl.pallas_call(
        functools.partial(_matmul_body, N),
        out_shape=jax.ShapeDtypeStruct((Np, H), jnp.float32),
    )(x, W1)

    (histo,) = _make_k1(N, E, Np)(dst2)
    ysh, part1, diso = _make_k2(N, E, H, Np)(xw, src2, dst2, histo)
    ysh2, part2 = _make_k3(N, E, H, Np)(src2, dst2, part1, ysh, diso, b1)

    out = pl.pallas_call(
        _final_body,
        out_shape=jax.ShapeDtypeStruct((Np, C), jnp.float32),
    )(part2, W2, b2.reshape(1, C))
    return out[:N]
